# Initial kernel scaffold; baseline (speedup 1.0000x reference)
import jax
import jax.numpy as jnp
from jax.experimental import pallas as pl


def kernel(x):
    raise NotImplementedError("write your pallas kernel here")


# speedup vs baseline: 156.1879x; 156.1879x over previous
import functools

import numpy as np
import jax
import jax.numpy as jnp
from jax import lax
from jax.experimental import pallas as pl
from jax.experimental.pallas import tpu as pltpu
from jax.experimental.pallas import tpu_sc as plsc

_L = 16

_CABINP1 = np.array([[-20.0, 3.0], [-10.0, 2.0], [0.0, 0.88], [10.0, 0.38],
                     [25.0, 0.7], [30.0, 1.0], [35.0, 1.31], [40.0, 2.5],
                     [45.0, 3.0]], dtype=np.float64)
_CABINP2 = np.array([[-20.0, 3.0], [-15.0, 2.0], [-10.0, 1.0], [-5.0, 0.5],
                     [0.0, 0.4], [5.0, 0.5], [10.0, 1.5], [15.0, 3.0],
                     [20.0, 6.0]], dtype=np.float64)
_TEMP_SET = np.array([18.0, 20, 22, 24, 26, 28, 30, 31.5, 32], dtype=np.float64)
_TEMP_ENVR = np.array([-30.0, -20, -10, 0, 5, 10, 15, 20, 25, 30, 35, 40, 45, 50],
                      dtype=np.float64)
_CABINSP = np.array([
    [17.0, 17, 17, 17, 17, 17, 17, 17, 17, 17, 17, 17, 17, 17],
    [20, 20, 19.5, 19.5, 19.5, 19, 19, 19, 18.5, 18.5, 18, 18, 18, 18],
    [22, 22, 22, 22.5, 22.5, 22.5, 22, 22, 21, 21, 21, 21, 20.5, 20],
    [24, 24.5, 25.5, 25.5, 26, 26, 25.5, 25, 24.5, 24, 23.5, 23, 23, 23],
    [27, 26.5, 27, 27.5, 28, 28, 27.5, 27, 26.5, 26, 25.5, 26, 26, 26],
    [29, 28.5, 28.5, 29.5, 30, 30, 29.5, 29, 29, 29, 28, 28, 29, 29],
    [31, 30.5, 30.5, 31.5, 32, 32, 32, 31, 31, 31, 31, 31, 31, 31],
    [32, 32, 32, 33, 33, 33, 33, 33, 33, 33, 33, 33, 32, 32],
    [32, 32, 36, 36, 36, 36, 36, 36, 36, 36, 36, 36, 36, 36]], dtype=np.float64)


def _ramp1d(xp, fp):
    s = np.diff(fp) / np.diff(xp)
    lo, hi = xp[:-1], xp[1:]
    c0 = fp[0] - np.sum(s * lo)
    return ([float(v) for v in s], [float(v) for v in lo],
            [float(v) for v in hi], float(c0))

_P1 = _ramp1d(_CABINP1[:, 0], _CABINP1[:, 1])
_P2 = _ramp1d(_CABINP2[:, 0], _CABINP2[:, 1])
_S2 = np.diff(_CABINSP, axis=1) / np.diff(_TEMP_ENVR)[None, :]
_C2 = _CABINSP[:, 0] - (_S2 * _TEMP_ENVR[:-1][None, :]).sum(axis=1)
_CLO = [float(v) for v in _TEMP_ENVR[:-1]]
_CHI = [float(v) for v in _TEMP_ENVR[1:]]
_RA = [float(v) for v in 1.0 / np.diff(_TEMP_SET)]
_RB = [float(v) for v in -_TEMP_SET[:-1] / np.diff(_TEMP_SET)]


def _interp1(q, params):
    s, lo, hi, c0 = params
    acc = jnp.full((_L,), np.float32(c0), jnp.float32)
    for k in range(len(s)):
        acc = acc + np.float32(s[k]) * jnp.clip(q, np.float32(lo[k]),
                                                np.float32(hi[k]))
    return acc


def _group_compute(a, fl, fr, ti):
    kp1 = _interp1(a, _P1)
    m = [jnp.clip(a, np.float32(_CLO[k]), np.float32(_CHI[k]))
         for k in range(13)]
    G = []
    for i in range(9):
        acc = jnp.full((_L,), np.float32(_C2[i]), jnp.float32)
        for k in range(13):
            acc = acc + np.float32(_S2[i, k]) * m[k]
        G.append(acc)
    dG = [G[i + 1] - G[i] for i in range(8)]

    def row_interp(q):
        acc = G[0]
        for k in range(8):
            u = jnp.clip(q * np.float32(_RA[k]) + np.float32(_RB[k]),
                         np.float32(0.0), np.float32(1.0))
            acc = acc + dG[k] * u
        return acc

    err = jnp.minimum(row_interp(fl), row_interp(fr)) - ti
    kp2 = _interp1(err, _P2)
    return jnp.minimum(kp1, kp2)


@functools.cache
def _make_sc_kernel(nrows, ncols):
    NW = 32
    R = nrows // NW
    CH = 2048
    NCH = R // CH
    NPAIR = NCH // 2
    CHW = CH * ncols
    GRP = CH // _L
    mesh = plsc.VectorSubcoreMesh(core_axis_name="c", subcore_axis_name="s")

    @functools.partial(
        pl.kernel, mesh=mesh,
        compiler_params=pltpu.CompilerParams(needs_layout_passes=False),
        out_type=jax.ShapeDtypeStruct((nrows,), jnp.float32),
        scratch_types=[
            pltpu.VMEM((CHW,), jnp.float32),
            pltpu.VMEM((CHW,), jnp.float32),
            pltpu.VMEM((CH,), jnp.float32),
            pltpu.VMEM((CH,), jnp.float32),
            pltpu.SemaphoreType.DMA,
            pltpu.SemaphoreType.DMA,
        ],
    )
    def sc_kernel(x_hbm, out_hbm, xv0, xv1, ov0, ov1, si0, si1):
        wid = lax.axis_index("s") * 2 + lax.axis_index("c")
        base_row = wid * R
        base_flat = base_row * ncols

        def in_cp(ch, xv, sem):
            return pltpu.make_async_copy(
                x_hbm.at[pl.ds(base_flat + ch * CHW, CHW)], xv, sem)

        def compute(xv, ov):
            i16 = lax.broadcasted_iota(jnp.int32, (_L,), 0) * ncols

            def body(g, carry):
                b = g * (_L * ncols)
                a = plsc.load_gather(xv, [i16 + (b + 1)])
                fl = plsc.load_gather(xv, [i16 + (b + 2)])
                fr = plsc.load_gather(xv, [i16 + (b + 3)])
                ti = plsc.load_gather(xv, [i16 + (b + 8)])
                ov[pl.ds(g * _L, _L)] = _group_compute(a, fl, fr, ti)
                return carry

            lax.fori_loop(0, GRP, body, 0)

        in_cp(0, xv0, si0).start()
        in_cp(1, xv1, si1).start()

        def pair(p, carry):
            chA = p * 2
            in_cp(chA, xv0, si0).wait()
            compute(xv0, ov0)

            @pl.when(p < NPAIR - 1)
            def _():
                in_cp(chA + 2, xv0, si0).start()

            pltpu.sync_copy(ov0, out_hbm.at[pl.ds(base_row + chA * CH, CH)])

            in_cp(chA + 1, xv1, si1).wait()
            compute(xv1, ov1)

            @pl.when(p < NPAIR - 1)
            def _():
                in_cp(chA + 3, xv1, si1).start()

            pltpu.sync_copy(ov1,
                            out_hbm.at[pl.ds(base_row + (chA + 1) * CH, CH)])
            return carry

        lax.fori_loop(0, NPAIR, pair, 0)

    return sc_kernel


def kernel(x):
    if x.ndim == 1:
        x = x[None, :]
    nrows, ncols = x.shape
    out = _make_sc_kernel(nrows, ncols)(x.reshape(-1))
    return out.reshape(-1, 1)
